# baseline (speedup 1.0000x reference)
import functools

import jax
import jax.numpy as jnp
from jax import lax
from jax.experimental import pallas as pl
from jax.experimental.pallas import tpu as pltpu
from jax.experimental.pallas import tpu_sc as plsc

BATCH = 65536
INP = 65536
OUT = 64
NNZ = 4194304

NC = 2
NS = 16
NW = NC * NS

CHUNK = 512
NCHUNK = NNZ // CHUNK
R = 512
NB = BATCH // R
NH = CHUNK // 128
NBW = NB // NW


def _sc_body(vals2d, rows2d, cols3d, w_hbm, bnds, out_hbm,
             acc, g_v, cols_v, rows_v, vals_v, bnd_v, isem, gsem):
    wid = lax.axis_index("s") * NC + lax.axis_index("c")

    pltpu.sync_copy(bnds, bnd_v)
    bvec = bnd_v[pl.ds(wid * NBW, 16)]

    def issue_idx(ck):
        b = lax.rem(ck, 3)
        sl = pl.ds(ck * CHUNK, CHUNK)
        pltpu.async_copy(cols3d.at[ck], cols_v.at[b], isem.at[0, b])
        pltpu.async_copy(rows2d.at[sl], rows_v.at[b], isem.at[1, b])
        pltpu.async_copy(vals2d.at[sl], vals_v.at[b], isem.at[2, b])

    def wait_cols(ck):
        b = lax.rem(ck, 3)
        pltpu.make_async_copy(cols3d.at[ck], cols_v.at[b], isem.at[0, b]).wait()

    def wait_rv(ck):
        b = lax.rem(ck, 3)
        sl = pl.ds(ck * CHUNK, CHUNK)
        pltpu.make_async_copy(rows2d.at[sl], rows_v.at[b], isem.at[1, b]).wait()
        pltpu.make_async_copy(vals2d.at[sl], vals_v.at[b], isem.at[2, b]).wait()

    def issue_gather(ck):
        b2 = lax.rem(ck, 2)
        b3 = lax.rem(ck, 3)
        for h in range(NH):
            pltpu.async_copy(
                w_hbm.at[cols_v.at[b3, h]],
                g_v.at[b2, pl.ds(h * 128, 128)],
                gsem.at[b2, h],
            )

    def wait_gather(ck):
        b2 = lax.rem(ck, 2)
        b3 = lax.rem(ck, 3)
        for h in range(NH):
            pltpu.make_async_copy(
                w_hbm.at[cols_v.at[b3, h]],
                g_v.at[b2, pl.ds(h * 128, 128)],
                gsem.at[b2, h],
            ).wait()

    for j in range(NBW):
        e_lo = bvec[j]
        e_hi = bvec[j + 1]
        base = (wid * NBW + j) * R

        c_lo = e_lo // CHUNK
        c_hi = (e_hi + CHUNK - 1) // CHUNK
        n = c_hi - c_lo

        pl.when(n > 0)(lambda: issue_idx(c_lo))
        pl.when(n > 1)(lambda: issue_idx(c_lo + 1))

        def zbody(k, _):
            acc[pl.ds(k * 16, 16)] = jnp.zeros((16,), jnp.float32)
            return 0
        lax.fori_loop(0, R * OUT // 16, zbody, 0)

        def first_gather():
            wait_cols(c_lo)
            issue_gather(c_lo)
        pl.when(n > 0)(first_gather)

        def cbody(ck, _):
            pl.when(ck + 2 < c_hi)(lambda: issue_idx(ck + 2))

            def next_gather():
                wait_cols(ck + 1)
                issue_gather(ck + 1)
            pl.when(ck + 1 < c_hi)(next_gather)

            wait_gather(ck)
            wait_rv(ck)
            b2 = lax.rem(ck, 2)
            b3 = lax.rem(ck, 3)

            @plsc.parallel_loop(0, CHUNK // 16, unroll=4)
            def gbody(g16):
                rows16 = rows_v[b3, pl.ds(g16 * 16, 16)]
                vals16 = vals_v[b3, pl.ds(g16 * 16, 16)]
                lr16 = rows16 - base
                ok16 = (lr16 >= 0) & (lr16 < R)
                lrc = jnp.where(ok16, lr16, 0)
                vv = jnp.where(ok16, vals16, jnp.float32(0))
                for jj in range(0, 16, 4):
                    gvs = []
                    for t in range(4):
                        for h in range(OUT // 32):
                            raw = g_v[
                                b2, g16 * 16 + jj + t, pl.ds(32 * h, 32)
                            ]
                            a, b = plsc.unpack(
                                raw,
                                format=plsc.PackFormat.INTERLEAVED,
                                preferred_element_type=jnp.float32,
                            )
                            gvs += [a, b]
                    for t in range(4):
                        off = lrc[jj + t] * OUT
                        v = vv[jj + t]
                        for q in range(OUT // 16):
                            plsc.addupdate(
                                acc.at[pl.ds(off + 16 * q, 16)],
                                gvs[t * (OUT // 16) + q] * v,
                            )
            return 0
        lax.fori_loop(c_lo, c_hi, cbody, 0)

        pltpu.sync_copy(acc, out_hbm.at[pl.ds(base * OUT, R * OUT)])


@jax.jit
def _sc_call(vals2d, rows2d, cols2d, W, bnds):
    mesh = plsc.VectorSubcoreMesh(
        core_axis_name="c", subcore_axis_name="s", num_cores=NC, num_subcores=NS
    )
    return pl.kernel(
        _sc_body,
        out_type=jax.ShapeDtypeStruct((BATCH * OUT,), jnp.float32),
        mesh=mesh,
        compiler_params=pltpu.CompilerParams(
            use_tc_tiling_on_sc=False, needs_layout_passes=False
        ),
        scratch_types=[
            pltpu.VMEM((R * OUT,), jnp.float32),
            pltpu.VMEM((2, CHUNK, OUT), jnp.bfloat16),
            pltpu.VMEM((3, NH, 128), jnp.int32),
            pltpu.VMEM((3, CHUNK), jnp.int32),
            pltpu.VMEM((3, CHUNK), jnp.float32),
            pltpu.VMEM((NB + 16,), jnp.int32),
            pltpu.SemaphoreType.DMA((3, 3)),
            pltpu.SemaphoreType.DMA((2, NH)),
        ],
    )(vals2d, rows2d, cols2d, W, bnds)


def kernel(values, row_indices, col_indices, W):
    rows = row_indices.astype(jnp.int32)
    cols = col_indices.astype(jnp.int32)
    pos = jnp.arange(OUT)
    src_col = (pos // 32) * 32 + (pos % 2) * 16 + (pos % 32) // 2
    W16 = W.astype(jnp.bfloat16)[:, src_col]
    edges = jnp.arange(0, BATCH + 1, R, dtype=jnp.int32)
    bounds = jnp.searchsorted(rows, edges, method="compare_all").astype(jnp.int32)
    bnds = jnp.concatenate([bounds, jnp.zeros((15,), jnp.int32)])
    out = _sc_call(values, rows, cols.reshape(NCHUNK, NH, 128), W16, bnds)
    return out.reshape(BATCH, OUT)


# speedup vs baseline: 1.4058x; 1.4058x over previous
import functools

import jax
import jax.numpy as jnp
from jax import lax
from jax.experimental import pallas as pl
from jax.experimental.pallas import tpu as pltpu
from jax.experimental.pallas import tpu_sc as plsc

BATCH = 65536
INP = 65536
OUT = 64
NNZ = 4194304

NC = 2
NS = 16
NW = NC * NS

CHUNK = 512
NCHUNK = NNZ // CHUNK
R = 512
NB = BATCH // R
NH = CHUNK // 128
NBW = NB // NW


def _sc_body(vals2d, rows2d, cols3d, w_hbm, bnds, out_hbm,
             acc, g_v, cols_v, rows_v, vals_v, bnd_v, isem, gsem):
    wid = lax.axis_index("s") * NC + lax.axis_index("c")

    pltpu.sync_copy(bnds, bnd_v)
    bvec = bnd_v[pl.ds(wid * NBW, 16)]

    def issue_idx(ck):
        b = lax.rem(ck, 3)
        sl = pl.ds(ck * CHUNK, CHUNK)
        pltpu.async_copy(cols3d.at[ck], cols_v.at[b], isem.at[0, b])
        pltpu.async_copy(rows2d.at[sl], rows_v.at[b], isem.at[1, b])
        pltpu.async_copy(vals2d.at[sl], vals_v.at[b], isem.at[2, b])

    def wait_cols(ck):
        b = lax.rem(ck, 3)
        pltpu.make_async_copy(cols3d.at[ck], cols_v.at[b], isem.at[0, b]).wait()

    def wait_rv(ck):
        b = lax.rem(ck, 3)
        sl = pl.ds(ck * CHUNK, CHUNK)
        pltpu.make_async_copy(rows2d.at[sl], rows_v.at[b], isem.at[1, b]).wait()
        pltpu.make_async_copy(vals2d.at[sl], vals_v.at[b], isem.at[2, b]).wait()

    def issue_gather(ck):
        b2 = lax.rem(ck, 2)
        b3 = lax.rem(ck, 3)
        for h in range(NH):
            pltpu.async_copy(
                w_hbm.at[cols_v.at[b3, h]],
                g_v.at[b2, pl.ds(h * 128, 128)],
                gsem.at[b2, h],
            )

    def wait_gather(ck):
        b2 = lax.rem(ck, 2)
        b3 = lax.rem(ck, 3)
        for h in range(NH):
            pltpu.make_async_copy(
                w_hbm.at[cols_v.at[b3, h]],
                g_v.at[b2, pl.ds(h * 128, 128)],
                gsem.at[b2, h],
            ).wait()

    for j in range(NBW):
        e_lo = bvec[j]
        e_hi = bvec[j + 1]
        base = (wid * NBW + j) * R

        c_lo = e_lo // CHUNK
        c_hi = (e_hi + CHUNK - 1) // CHUNK
        n = c_hi - c_lo

        pl.when(n > 0)(lambda: issue_idx(c_lo))
        pl.when(n > 1)(lambda: issue_idx(c_lo + 1))

        def zbody(k, _):
            acc[pl.ds(k * 16, 16)] = jnp.zeros((16,), jnp.float32)
            return 0
        lax.fori_loop(0, R * OUT // 16, zbody, 0)

        def first_gather():
            wait_cols(c_lo)
            issue_gather(c_lo)
        pl.when(n > 0)(first_gather)

        def cbody(ck, _):
            pl.when(ck + 2 < c_hi)(lambda: issue_idx(ck + 2))

            def next_gather():
                wait_cols(ck + 1)
                issue_gather(ck + 1)
            pl.when(ck + 1 < c_hi)(next_gather)

            wait_gather(ck)
            wait_rv(ck)
            b2 = lax.rem(ck, 2)
            b3 = lax.rem(ck, 3)

            @plsc.parallel_loop(0, CHUNK // 16, unroll=4)
            def gbody(g16):
                rows16 = rows_v[b3, pl.ds(g16 * 16, 16)]
                vals16 = vals_v[b3, pl.ds(g16 * 16, 16)]
                lr16 = rows16 - base
                ok16 = (lr16 >= 0) & (lr16 < R)
                lrc = jnp.where(ok16, lr16, 0)
                vv = jnp.where(ok16, vals16, jnp.float32(0))
                for jj in range(0, 16, 4):
                    gvs = []
                    for t in range(4):
                        for h in range(OUT // 32):
                            raw = g_v[
                                b2, g16 * 16 + jj + t, pl.ds(32 * h, 32)
                            ]
                            a, b = plsc.unpack(
                                raw,
                                format=plsc.PackFormat.INTERLEAVED,
                                preferred_element_type=jnp.float32,
                            )
                            gvs += [a, b]
                    for t in range(4):
                        off = lrc[jj + t] * OUT
                        v = vv[jj + t]
                        for q in range(OUT // 16):
                            plsc.addupdate(
                                acc.at[pl.ds(off + 16 * q, 16)],
                                gvs[t * (OUT // 16) + q] * v,
                            )
            return 0
        lax.fori_loop(c_lo, c_hi, cbody, 0)

        pltpu.sync_copy(acc, out_hbm.at[pl.ds(base * OUT, R * OUT)])


@jax.jit
def _sc_call(vals2d, rows2d, cols2d, W, bnds):
    mesh = plsc.VectorSubcoreMesh(
        core_axis_name="c", subcore_axis_name="s", num_cores=NC, num_subcores=NS
    )
    return pl.kernel(
        _sc_body,
        out_type=jax.ShapeDtypeStruct((BATCH * OUT,), jnp.float32),
        mesh=mesh,
        compiler_params=pltpu.CompilerParams(
            use_tc_tiling_on_sc=False, needs_layout_passes=False
        ),
        scratch_types=[
            pltpu.VMEM((R * OUT,), jnp.float32),
            pltpu.VMEM((2, CHUNK, OUT), jnp.bfloat16),
            pltpu.VMEM((3, NH, 128), jnp.int32),
            pltpu.VMEM((3, CHUNK), jnp.int32),
            pltpu.VMEM((3, CHUNK), jnp.float32),
            pltpu.VMEM((NB + 16,), jnp.int32),
            pltpu.SemaphoreType.DMA((3, 3)),
            pltpu.SemaphoreType.DMA((2, NH)),
        ],
    )(vals2d, rows2d, cols2d, W, bnds)


def kernel(values, row_indices, col_indices, W):
    rows = row_indices.astype(jnp.int32)
    cols = col_indices.astype(jnp.int32)
    pos = jnp.arange(OUT)
    src_col = (pos // 32) * 32 + (pos % 2) * 16 + (pos % 32) // 2
    W16 = W.astype(jnp.bfloat16)[:, src_col]
    edges = jnp.arange(0, BATCH + 1, R, dtype=jnp.int32)
    bounds = jnp.searchsorted(rows, edges).astype(jnp.int32)
    bnds = jnp.concatenate([bounds, jnp.zeros((15,), jnp.int32)])
    out = _sc_call(values, rows, cols.reshape(NCHUNK, NH, 128), W16, bnds)
    return out.reshape(BATCH, OUT)


# speedup vs baseline: 1.4647x; 1.0419x over previous
import functools

import jax
import jax.numpy as jnp
from jax import lax
from jax.experimental import pallas as pl
from jax.experimental.pallas import tpu as pltpu
from jax.experimental.pallas import tpu_sc as plsc

BATCH = 65536
INP = 65536
OUT = 64
NNZ = 4194304

NC = 2
NS = 16
NW = NC * NS

CHUNK = 512
NCHUNK = NNZ // CHUNK
R = 512
NB = BATCH // R
NH = CHUNK // 128
NBW = NB // NW


def _sc_body(vals2d, rows2d, cols3d, w_hbm, bnds, out_hbm,
             acc, g_v, cols_v, rows_v, vals_v, bnd_v, isem, gsem):
    wid = lax.axis_index("s") * NC + lax.axis_index("c")

    pltpu.sync_copy(bnds, bnd_v)
    bvec = bnd_v[pl.ds(wid * NBW, 16)]

    def issue_idx(ck):
        b = lax.rem(ck, 3)
        sl = pl.ds(ck * CHUNK, CHUNK)
        pltpu.async_copy(cols3d.at[ck], cols_v.at[b], isem.at[0, b])
        pltpu.async_copy(rows2d.at[sl], rows_v.at[b], isem.at[1, b])
        pltpu.async_copy(vals2d.at[sl], vals_v.at[b], isem.at[2, b])

    def wait_cols(ck):
        b = lax.rem(ck, 3)
        pltpu.make_async_copy(cols3d.at[ck], cols_v.at[b], isem.at[0, b]).wait()

    def wait_rv(ck):
        b = lax.rem(ck, 3)
        sl = pl.ds(ck * CHUNK, CHUNK)
        pltpu.make_async_copy(rows2d.at[sl], rows_v.at[b], isem.at[1, b]).wait()
        pltpu.make_async_copy(vals2d.at[sl], vals_v.at[b], isem.at[2, b]).wait()

    def issue_gather(ck):
        b2 = lax.rem(ck, 2)
        b3 = lax.rem(ck, 3)
        for h in range(NH):
            pltpu.async_copy(
                w_hbm.at[cols_v.at[b3, h]],
                g_v.at[b2, pl.ds(h * 128, 128)],
                gsem.at[b2, h],
            )

    def wait_gather(ck):
        b2 = lax.rem(ck, 2)
        b3 = lax.rem(ck, 3)
        for h in range(NH):
            pltpu.make_async_copy(
                w_hbm.at[cols_v.at[b3, h]],
                g_v.at[b2, pl.ds(h * 128, 128)],
                gsem.at[b2, h],
            ).wait()

    for j in range(NBW):
        e_lo = bvec[j]
        e_hi = bvec[j + 1]
        base = (wid * NBW + j) * R

        c_lo = e_lo // CHUNK
        c_hi = (e_hi + CHUNK - 1) // CHUNK
        n = c_hi - c_lo

        pl.when(n > 0)(lambda: issue_idx(c_lo))
        pl.when(n > 1)(lambda: issue_idx(c_lo + 1))

        def first_gather():
            wait_cols(c_lo)
            issue_gather(c_lo)
        pl.when(n > 0)(first_gather)

        z16 = jnp.zeros((16,), jnp.float32)

        @plsc.parallel_loop(0, R * OUT // 128, unroll=2)
        def zbody(k):
            for u in range(8):
                acc[pl.ds(k * 128 + u * 16, 16)] = z16

        def cbody(ck, _):
            pl.when(ck + 2 < c_hi)(lambda: issue_idx(ck + 2))

            def next_gather():
                wait_cols(ck + 1)
                issue_gather(ck + 1)
            pl.when(ck + 1 < c_hi)(next_gather)

            wait_gather(ck)
            wait_rv(ck)
            b2 = lax.rem(ck, 2)
            b3 = lax.rem(ck, 3)

            @plsc.parallel_loop(0, CHUNK // 16, unroll=4)
            def gbody(g16):
                rows16 = rows_v[b3, pl.ds(g16 * 16, 16)]
                vals16 = vals_v[b3, pl.ds(g16 * 16, 16)]
                lr16 = rows16 - base
                ok16 = (lr16 >= 0) & (lr16 < R)
                lrc = jnp.where(ok16, lr16, 0)
                vv = jnp.where(ok16, vals16, jnp.float32(0))
                for jj in range(0, 16, 4):
                    gvs = []
                    for t in range(4):
                        for h in range(OUT // 32):
                            raw = g_v[
                                b2, g16 * 16 + jj + t, pl.ds(32 * h, 32)
                            ]
                            a, b = plsc.unpack(
                                raw,
                                format=plsc.PackFormat.INTERLEAVED,
                                preferred_element_type=jnp.float32,
                            )
                            gvs += [a, b]
                    for t in range(4):
                        off = lrc[jj + t] * OUT
                        v = vv[jj + t]
                        for q in range(OUT // 16):
                            plsc.addupdate(
                                acc.at[pl.ds(off + 16 * q, 16)],
                                gvs[t * (OUT // 16) + q] * v,
                            )
            return 0
        lax.fori_loop(c_lo, c_hi, cbody, 0)

        pltpu.sync_copy(acc, out_hbm.at[pl.ds(base * OUT, R * OUT)])


@jax.jit
def _sc_call(vals2d, rows2d, cols2d, W, bnds):
    mesh = plsc.VectorSubcoreMesh(
        core_axis_name="c", subcore_axis_name="s", num_cores=NC, num_subcores=NS
    )
    return pl.kernel(
        _sc_body,
        out_type=jax.ShapeDtypeStruct((BATCH * OUT,), jnp.float32),
        mesh=mesh,
        compiler_params=pltpu.CompilerParams(
            use_tc_tiling_on_sc=False, needs_layout_passes=False
        ),
        scratch_types=[
            pltpu.VMEM((R * OUT,), jnp.float32),
            pltpu.VMEM((2, CHUNK, OUT), jnp.bfloat16),
            pltpu.VMEM((3, NH, 128), jnp.int32),
            pltpu.VMEM((3, CHUNK), jnp.int32),
            pltpu.VMEM((3, CHUNK), jnp.float32),
            pltpu.VMEM((NB + 16,), jnp.int32),
            pltpu.SemaphoreType.DMA((3, 3)),
            pltpu.SemaphoreType.DMA((2, NH)),
        ],
    )(vals2d, rows2d, cols2d, W, bnds)


def kernel(values, row_indices, col_indices, W):
    rows = row_indices.astype(jnp.int32)
    cols = col_indices.astype(jnp.int32)
    pos = jnp.arange(OUT)
    src_col = (pos // 32) * 32 + (pos % 2) * 16 + (pos % 32) // 2
    W16 = W.astype(jnp.bfloat16)[:, src_col]
    edges = jnp.arange(0, BATCH + 1, R, dtype=jnp.int32)
    bounds = jnp.searchsorted(rows, edges).astype(jnp.int32)
    bnds = jnp.concatenate([bounds, jnp.zeros((15,), jnp.int32)])
    out = _sc_call(values, rows, cols.reshape(NCHUNK, NH, 128), W16, bnds)
    return out.reshape(BATCH, OUT)
